# baseline (device time: 103471 ns/iter reference)
import jax
import jax.numpy as jnp
from jax import lax
from jax.experimental import pallas as pl
from jax.experimental.pallas import tpu as pltpu

N_DEV = 4
M = 2048
D = 2048
C = M // N_DEV
H = C // 2


def kernel(partial, resid, gamma):

    def body(x_ref, resid_hbm, gamma_ref, out_ref,
             sendA, recvA, accA, acc2A,
             sendB, recvB, accB, acc2B,
             resid_v, s_sem, r_sem):
        p = lax.axis_index("i")
        q1 = p ^ 1
        q2 = 3 - p

        def arows(ck):
            return pl.ds(ck * C, H)

        def brows(ck):
            return pl.ds(ck * C + H, H)

        def xa(ck):
            return x_ref[0, arows(ck), :]

        def xb(ck):
            return x_ref[0, brows(ck), :]

        def xchg(i, src, dst, tgt):
            rd = pltpu.make_async_remote_copy(
                src_ref=src, dst_ref=dst,
                send_sem=s_sem.at[i], recv_sem=r_sem.at[i],
                device_id=(tgt,), device_id_type=pl.DeviceIdType.MESH)
            rd.start()
            return rd

        barrier = pltpu.get_barrier_semaphore()
        for nbr in (q1, q2):
            pl.semaphore_signal(barrier, inc=1, device_id=(nbr,),
                                device_id_type=pl.DeviceIdType.MESH)
        pl.semaphore_wait(barrier, 2)

        sendA[0] = xa(q1).astype(jnp.bfloat16)
        sendA[1] = xa(3 - q1).astype(jnp.bfloat16)
        rs1a = xchg(0, sendA, recvA, q1)
        sendB[0] = xb(q2).astype(jnp.bfloat16)
        sendB[1] = xb(q2 ^ 1).astype(jnp.bfloat16)
        rs1b = xchg(1, sendB, recvB, q2)
        resid_cp = pltpu.make_async_copy(
            resid_hbm.at[pl.ds(p * C, C), :], resid_v, s_sem.at[8])
        resid_cp.start()
        accA[0] = xa(p).astype(jnp.bfloat16)
        accA[1] = xa(3 - p).astype(jnp.bfloat16)
        accB[0] = xb(p).astype(jnp.bfloat16)
        accB[1] = xb(p ^ 1).astype(jnp.bfloat16)

        rs1a.wait()
        accA[0] = accA[0] + recvA[0]
        accA[1] = accA[1] + recvA[1]
        rs2a = xchg(2, accA.at[1], acc2A, q2)
        rs1b.wait()
        accB[0] = accB[0] + recvB[0]
        accB[1] = accB[1] + recvB[1]
        rs2b = xchg(3, accB.at[1], acc2B, q1)

        agA, gA = sendA, recvA
        agB, gB = sendB, recvB
        rs2a.wait()
        resid_cp.wait()
        ya = (accA[0].astype(jnp.float32) + acc2A[:, :].astype(jnp.float32)
              + resid_v[0:H, :])
        ra = jnp.sqrt(jnp.mean(ya * ya, axis=-1, keepdims=True) + 1e-6)
        oa = ya / ra * gamma_ref[:]
        agA[0] = oa.astype(jnp.bfloat16)
        ag1a = xchg(4, agA.at[0], agA.at[1], q2)
        out_ref[arows(p), :] = oa

        rs2b.wait()
        yb = (accB[0].astype(jnp.float32) + acc2B[:, :].astype(jnp.float32)
              + resid_v[H:C, :])
        rb = jnp.sqrt(jnp.mean(yb * yb, axis=-1, keepdims=True) + 1e-6)
        ob = yb / rb * gamma_ref[:]
        agB[0] = ob.astype(jnp.bfloat16)
        ag1b = xchg(5, agB.at[0], agB.at[1], q1)
        out_ref[brows(p), :] = ob

        ag1a.wait()
        ag2a = xchg(6, agA, gA, q1)
        out_ref[arows(3 - p), :] = agA[1].astype(jnp.float32)
        ag1b.wait()
        ag2b = xchg(7, agB, gB, q2)
        out_ref[brows(p ^ 1), :] = agB[1].astype(jnp.float32)

        ag2a.wait()
        out_ref[arows(q1), :] = gA[0].astype(jnp.float32)
        out_ref[arows(3 - q1), :] = gA[1].astype(jnp.float32)
        ag2b.wait()
        out_ref[brows(q2), :] = gB[0].astype(jnp.float32)
        out_ref[brows(q2 ^ 1), :] = gB[1].astype(jnp.float32)

    bf = jnp.bfloat16
    return pl.pallas_call(
        body,
        out_shape=jax.ShapeDtypeStruct((M, D), jnp.float32),
        in_specs=[
            pl.BlockSpec(memory_space=pltpu.VMEM),
            pl.BlockSpec(memory_space=pl.ANY),
            pl.BlockSpec(memory_space=pltpu.VMEM),
        ],
        out_specs=pl.BlockSpec(memory_space=pltpu.VMEM),
        scratch_shapes=[
            pltpu.VMEM((2, H, D), bf),
            pltpu.VMEM((2, H, D), bf),
            pltpu.VMEM((2, H, D), bf),
            pltpu.VMEM((H, D), bf),
            pltpu.VMEM((2, H, D), bf),
            pltpu.VMEM((2, H, D), bf),
            pltpu.VMEM((2, H, D), bf),
            pltpu.VMEM((H, D), bf),
            pltpu.VMEM((C, D), jnp.float32),
            pltpu.SemaphoreType.DMA((9,)),
            pltpu.SemaphoreType.DMA((8,)),
        ],
        compiler_params=pltpu.CompilerParams(
            collective_id=0, vmem_limit_bytes=100 * 1024 * 1024),
    )(partial, resid, gamma)


# device time: 95723 ns/iter; 1.0809x vs baseline; 1.0809x over previous
import jax
import jax.numpy as jnp
from jax import lax
from jax.experimental import pallas as pl
from jax.experimental.pallas import tpu as pltpu

N_DEV = 4
M = 2048
D = 2048
C = M // N_DEV
H = C // 2
LANES = 2
S = H // LANES


def kernel(partial, resid, gamma):

    def body(x_ref, resid_hbm, gamma_ref, out_ref,
             rsR, rsL, agR, agL, resid_v,
             rsR_s, rsR_r, rsL_s, rsL_r,
             agR_s, agR_r, agL_s, agL_r, res_sem):
        p = lax.axis_index("i")
        left = lax.rem(p + N_DEV - 1, N_DEV)
        right = lax.rem(p + 1, N_DEV)

        rs_buf = (rsR, rsL)
        ag_buf = (agR, agL)
        rs_sem = ((rsR_s, rsR_r), (rsL_s, rsL_r))
        ag_sem = ((agR_s, agR_r), (agL_s, agL_r))
        tgt = (right, left)
        base = (0, H)

        def send_chunk(d, h):
            return lax.rem(p + 3 - h, N_DEV) if d == 0 else \
                lax.rem(p + 1 + h, N_DEV)

        def recv_chunk(d, h):
            return lax.rem(p + 2 - h, N_DEV) if d == 0 else \
                lax.rem(p + 2 + h, N_DEV)

        def ag_origin(d, h):
            return lax.rem(p + 3 - h, N_DEV) if d == 0 else \
                lax.rem(p + 1 + h, N_DEV)

        def rows(ck, d, l):
            return pl.ds(ck * C + base[d] + l * S, S)

        def xs(ck, d, l):
            return x_ref[0, rows(ck, d, l), :]

        def make_rdma(buf, sems, d, l, h):
            s_, r_ = h % 2, (h + 1) % 2
            return pltpu.make_async_remote_copy(
                src_ref=buf[d].at[s_, l], dst_ref=buf[d].at[r_, l],
                send_sem=sems[d][0].at[s_, l], recv_sem=sems[d][1].at[r_, l],
                device_id=(tgt[d],), device_id_type=pl.DeviceIdType.MESH)

        barrier = pltpu.get_barrier_semaphore()
        for nbr in (left, right):
            pl.semaphore_signal(barrier, inc=1, device_id=(nbr,),
                                device_id_type=pl.DeviceIdType.MESH)
        pl.semaphore_wait(barrier, 2)

        resid_cp = pltpu.make_async_copy(
            resid_hbm.at[pl.ds(p * C, C), :], resid_v, res_sem)
        resid_cp.start()

        rs_fly = {}
        ag_fly = {}
        for l in range(LANES):
            for d in (0, 1):
                ck = send_chunk(d, 0)
                rs_buf[d][0, l] = xs(ck, d, l).astype(jnp.bfloat16)
                rd = make_rdma(rs_buf, rs_sem, d, l, 0)
                rd.start()
                rs_fly[(d, l)] = rd

        for h in range(N_DEV - 1):
            r_ = (h + 1) % 2
            for l in range(LANES):
                for d in (0, 1):
                    rs_fly[(d, l)].wait()
                    if h < N_DEV - 2:
                        ck = recv_chunk(d, h)
                        rs_buf[d][r_, l] = (
                            rs_buf[d][r_, l]
                            + xs(ck, d, l).astype(jnp.bfloat16))
                        rd = make_rdma(rs_buf, rs_sem, d, l, h + 1)
                        rd.start()
                        rs_fly[(d, l)] = rd
                    else:
                        my = rows(p, d, l)
                        if (d, l) == (0, 0):
                            resid_cp.wait()
                        y = (rs_buf[d][r_, l].astype(jnp.float32)
                             + xs(p, d, l)
                             + resid_v[pl.ds(base[d] + l * S, S), :])
                        rms = jnp.sqrt(
                            jnp.mean(y * y, axis=-1, keepdims=True) + 1e-6)
                        o = y / rms * gamma_ref[:]
                        ag_buf[d][0, l] = o.astype(jnp.bfloat16)
                        rd = make_rdma(ag_buf, ag_sem, d, l, 0)
                        rd.start()
                        ag_fly[(d, l)] = rd
                        out_ref[my, :] = o

        for h in range(N_DEV - 1):
            r_ = (h + 1) % 2
            for l in range(LANES):
                for d in (0, 1):
                    ag_fly[(d, l)].wait()
                    if h < N_DEV - 2:
                        rd = make_rdma(ag_buf, ag_sem, d, l, h + 1)
                        rd.start()
                        ag_fly[(d, l)] = rd
                    org = ag_origin(d, h)
                    out_ref[rows(org, d, l), :] = (
                        ag_buf[d][r_, l].astype(jnp.float32))

    return pl.pallas_call(
        body,
        out_shape=jax.ShapeDtypeStruct((M, D), jnp.float32),
        in_specs=[
            pl.BlockSpec(memory_space=pltpu.VMEM),
            pl.BlockSpec(memory_space=pl.ANY),
            pl.BlockSpec(memory_space=pltpu.VMEM),
        ],
        out_specs=pl.BlockSpec(memory_space=pltpu.VMEM),
        scratch_shapes=[
            pltpu.VMEM((2, LANES, S, D), jnp.bfloat16),
            pltpu.VMEM((2, LANES, S, D), jnp.bfloat16),
            pltpu.VMEM((2, LANES, S, D), jnp.bfloat16),
            pltpu.VMEM((2, LANES, S, D), jnp.bfloat16),
            pltpu.VMEM((C, D), jnp.float32),
            pltpu.SemaphoreType.DMA((2, LANES)),
            pltpu.SemaphoreType.DMA((2, LANES)),
            pltpu.SemaphoreType.DMA((2, LANES)),
            pltpu.SemaphoreType.DMA((2, LANES)),
            pltpu.SemaphoreType.DMA((2, LANES)),
            pltpu.SemaphoreType.DMA((2, LANES)),
            pltpu.SemaphoreType.DMA((2, LANES)),
            pltpu.SemaphoreType.DMA((2, LANES)),
            pltpu.SemaphoreType.DMA,
        ],
        compiler_params=pltpu.CompilerParams(
            collective_id=0, vmem_limit_bytes=100 * 1024 * 1024),
    )(partial, resid, gamma)


# device time: 95202 ns/iter; 1.0869x vs baseline; 1.0055x over previous
import jax
import jax.numpy as jnp
from jax import lax
from jax.experimental import pallas as pl
from jax.experimental.pallas import tpu as pltpu

N_DEV = 4
M = 2048
D = 2048
C = M // N_DEV
H = C // 2
LANES = 2
S = H // LANES


def kernel(partial, resid, gamma):

    def body(x_hbm, resid_hbm, gamma_ref, out_ref,
             rsR, rsL, agR, agL, resid_v, xv,
             rsR_s, rsR_r, rsL_s, rsL_r,
             agR_s, agR_r, agL_s, agL_r, res_sem, x_sem):
        p = lax.axis_index("i")
        left = lax.rem(p + N_DEV - 1, N_DEV)
        right = lax.rem(p + 1, N_DEV)

        rs_buf = (rsR, rsL)
        ag_buf = (agR, agL)
        rs_sem = ((rsR_s, rsR_r), (rsL_s, rsL_r))
        ag_sem = ((agR_s, agR_r), (agL_s, agL_r))
        tgt = (right, left)
        base = (0, H)

        def send_chunk(d, h):
            return lax.rem(p + 3 - h, N_DEV) if d == 0 else \
                lax.rem(p + 1 + h, N_DEV)

        def recv_chunk(d, h):
            return lax.rem(p + 2 - h, N_DEV) if d == 0 else \
                lax.rem(p + 2 + h, N_DEV)

        def ag_origin(d, h):
            return lax.rem(p + 3 - h, N_DEV) if d == 0 else \
                lax.rem(p + 1 + h, N_DEV)

        def rows(ck, d, l):
            return pl.ds(ck * C + base[d] + l * S, S)

        def xs(ck, d, l):
            return xv[rows(ck, d, l), :]

        def make_rdma(buf, sems, d, l, h):
            s_, r_ = h % 2, (h + 1) % 2
            return pltpu.make_async_remote_copy(
                src_ref=buf[d].at[s_, l], dst_ref=buf[d].at[r_, l],
                send_sem=sems[d][0].at[s_, l], recv_sem=sems[d][1].at[r_, l],
                device_id=(tgt[d],), device_id_type=pl.DeviceIdType.MESH)

        barrier = pltpu.get_barrier_semaphore()
        for nbr in (left, right):
            pl.semaphore_signal(barrier, inc=1, device_id=(nbr,),
                                device_id_type=pl.DeviceIdType.MESH)
        pl.semaphore_wait(barrier, 2)

        resid_cp = pltpu.make_async_copy(
            resid_hbm.at[pl.ds(p * C, C), :], resid_v, res_sem)
        resid_cp.start()

        x_order = (left, right, lax.rem(p + 2, N_DEV), p)
        x_cp = []
        for i, ck in enumerate(x_order):
            cp = pltpu.make_async_copy(
                x_hbm.at[0, pl.ds(ck * C, C), :],
                xv.at[pl.ds(ck * C, C), :], x_sem.at[i])
            cp.start()
            x_cp.append(cp)

        rs_fly = {}
        ag_fly = {}
        for l in range(LANES):
            for d in (0, 1):
                ck = send_chunk(d, 0)
                if l == 0:
                    x_cp[d].wait()
                rs_buf[d][0, l] = xs(ck, d, l).astype(jnp.bfloat16)
                rd = make_rdma(rs_buf, rs_sem, d, l, 0)
                rd.start()
                rs_fly[(d, l)] = rd

        for h in range(N_DEV - 1):
            r_ = (h + 1) % 2
            for l in range(LANES):
                for d in (0, 1):
                    rs_fly[(d, l)].wait()
                    if h == 0 and (d, l) == (0, 0):
                        x_cp[2].wait()
                    if h < N_DEV - 2:
                        ck = recv_chunk(d, h)
                        rs_buf[d][r_, l] = (
                            rs_buf[d][r_, l]
                            + xs(ck, d, l).astype(jnp.bfloat16))
                        rd = make_rdma(rs_buf, rs_sem, d, l, h + 1)
                        rd.start()
                        rs_fly[(d, l)] = rd
                    else:
                        my = rows(p, d, l)
                        if (d, l) == (0, 0):
                            x_cp[3].wait()
                            resid_cp.wait()
                        y = (rs_buf[d][r_, l].astype(jnp.float32)
                             + xs(p, d, l)
                             + resid_v[pl.ds(base[d] + l * S, S), :])
                        rms = jnp.sqrt(
                            jnp.mean(y * y, axis=-1, keepdims=True) + 1e-6)
                        o = y / rms * gamma_ref[:]
                        ag_buf[d][0, l] = o.astype(jnp.bfloat16)
                        rd = make_rdma(ag_buf, ag_sem, d, l, 0)
                        rd.start()
                        ag_fly[(d, l)] = rd
                        out_ref[my, :] = o

        for h in range(N_DEV - 1):
            r_ = (h + 1) % 2
            for l in range(LANES):
                for d in (0, 1):
                    ag_fly[(d, l)].wait()
                    if h < N_DEV - 2:
                        rd = make_rdma(ag_buf, ag_sem, d, l, h + 1)
                        rd.start()
                        ag_fly[(d, l)] = rd
                    org = ag_origin(d, h)
                    out_ref[rows(org, d, l), :] = (
                        ag_buf[d][r_, l].astype(jnp.float32))

    return pl.pallas_call(
        body,
        out_shape=jax.ShapeDtypeStruct((M, D), jnp.float32),
        in_specs=[
            pl.BlockSpec(memory_space=pl.ANY),
            pl.BlockSpec(memory_space=pl.ANY),
            pl.BlockSpec(memory_space=pltpu.VMEM),
        ],
        out_specs=pl.BlockSpec(memory_space=pltpu.VMEM),
        scratch_shapes=[
            pltpu.VMEM((2, LANES, S, D), jnp.bfloat16),
            pltpu.VMEM((2, LANES, S, D), jnp.bfloat16),
            pltpu.VMEM((2, LANES, S, D), jnp.bfloat16),
            pltpu.VMEM((2, LANES, S, D), jnp.bfloat16),
            pltpu.VMEM((C, D), jnp.float32),
            pltpu.VMEM((M, D), jnp.float32),
            pltpu.SemaphoreType.DMA((2, LANES)),
            pltpu.SemaphoreType.DMA((2, LANES)),
            pltpu.SemaphoreType.DMA((2, LANES)),
            pltpu.SemaphoreType.DMA((2, LANES)),
            pltpu.SemaphoreType.DMA((2, LANES)),
            pltpu.SemaphoreType.DMA((2, LANES)),
            pltpu.SemaphoreType.DMA((2, LANES)),
            pltpu.SemaphoreType.DMA((2, LANES)),
            pltpu.SemaphoreType.DMA,
            pltpu.SemaphoreType.DMA((4,)),
        ],
        compiler_params=pltpu.CompilerParams(
            collective_id=0, vmem_limit_bytes=100 * 1024 * 1024),
    )(partial, resid, gamma)


# device time: 90550 ns/iter; 1.1427x vs baseline; 1.0514x over previous
import jax
import jax.numpy as jnp
from jax import lax
from jax.experimental import pallas as pl
from jax.experimental.pallas import tpu as pltpu

N_DEV = 4
M = 2048
D = 2048
C = M // N_DEV
H = C // 2
LANES = 2
S = H // LANES


def kernel(partial, resid, gamma):

    def body(x_hbm, resid_hbm, gamma_ref, out_hbm,
             rsR, rsL, agR, agL, resid_v, xv, stg,
             rsR_s, rsR_r, rsL_s, rsL_r,
             agR_s, agR_r, agL_s, agL_r, res_sem, x_sem, o_sem):
        p = lax.axis_index("i")
        left = lax.rem(p + N_DEV - 1, N_DEV)
        right = lax.rem(p + 1, N_DEV)

        rs_buf = (rsR, rsL)
        ag_buf = (agR, agL)
        rs_sem = ((rsR_s, rsR_r), (rsL_s, rsL_r))
        ag_sem = ((agR_s, agR_r), (agL_s, agL_r))
        tgt = (right, left)
        base = (0, H)

        def send_chunk(d, h):
            return lax.rem(p + 3 - h, N_DEV) if d == 0 else \
                lax.rem(p + 1 + h, N_DEV)

        def recv_chunk(d, h):
            return lax.rem(p + 2 - h, N_DEV) if d == 0 else \
                lax.rem(p + 2 + h, N_DEV)

        def ag_origin(d, h):
            return lax.rem(p + 3 - h, N_DEV) if d == 0 else \
                lax.rem(p + 1 + h, N_DEV)

        def rows(ck, d, l):
            return pl.ds(ck * C + base[d] + l * S, S)

        def xs(ck, d, l):
            return xv[rows(ck, d, l), :]

        out_fly = {}

        def out_store(par, d, l, org, val):
            prev = out_fly.pop((par, d, l), None)
            if prev is not None:
                prev.wait()
            stg[par, d, l] = val
            cp = pltpu.make_async_copy(
                stg.at[par, d, l], out_hbm.at[rows(org, d, l), :],
                o_sem.at[par, d, l])
            cp.start()
            out_fly[(par, d, l)] = cp

        def make_rdma(buf, sems, d, l, h):
            s_, r_ = h % 2, (h + 1) % 2
            return pltpu.make_async_remote_copy(
                src_ref=buf[d].at[s_, l], dst_ref=buf[d].at[r_, l],
                send_sem=sems[d][0].at[s_, l], recv_sem=sems[d][1].at[r_, l],
                device_id=(tgt[d],), device_id_type=pl.DeviceIdType.MESH)

        barrier = pltpu.get_barrier_semaphore()
        for nbr in (left, right):
            pl.semaphore_signal(barrier, inc=1, device_id=(nbr,),
                                device_id_type=pl.DeviceIdType.MESH)
        pl.semaphore_wait(barrier, 2)

        resid_cp = pltpu.make_async_copy(
            resid_hbm.at[pl.ds(p * C, C), :], resid_v, res_sem)
        resid_cp.start()

        x_order = (left, right, lax.rem(p + 2, N_DEV), p)
        x_cp = []
        for i, ck in enumerate(x_order):
            cp = pltpu.make_async_copy(
                x_hbm.at[0, pl.ds(ck * C, C), :],
                xv.at[pl.ds(ck * C, C), :], x_sem.at[i])
            cp.start()
            x_cp.append(cp)

        rs_fly = {}
        ag_fly = {}
        for l in range(LANES):
            for d in (0, 1):
                ck = send_chunk(d, 0)
                if l == 0:
                    x_cp[d].wait()
                rs_buf[d][0, l] = xs(ck, d, l).astype(jnp.bfloat16)
                rd = make_rdma(rs_buf, rs_sem, d, l, 0)
                rd.start()
                rs_fly[(d, l)] = rd

        for h in range(N_DEV - 1):
            r_ = (h + 1) % 2
            for l in range(LANES):
                for d in (0, 1):
                    rs_fly[(d, l)].wait()
                    if h == 0 and (d, l) == (0, 0):
                        x_cp[2].wait()
                    if h < N_DEV - 2:
                        ck = recv_chunk(d, h)
                        rs_buf[d][r_, l] = (
                            rs_buf[d][r_, l]
                            + xs(ck, d, l).astype(jnp.bfloat16))
                        rd = make_rdma(rs_buf, rs_sem, d, l, h + 1)
                        rd.start()
                        rs_fly[(d, l)] = rd
                    else:
                        my = rows(p, d, l)
                        if (d, l) == (0, 0):
                            x_cp[3].wait()
                            resid_cp.wait()
                        y = (rs_buf[d][r_, l].astype(jnp.float32)
                             + xs(p, d, l)
                             + resid_v[pl.ds(base[d] + l * S, S), :])
                        rms = jnp.sqrt(
                            jnp.mean(y * y, axis=-1, keepdims=True) + 1e-6)
                        o = y / rms * gamma_ref[:]
                        ag_buf[d][0, l] = o.astype(jnp.bfloat16)
                        rd = make_rdma(ag_buf, ag_sem, d, l, 0)
                        rd.start()
                        ag_fly[(d, l)] = rd
                        out_store(0, d, l, p, o)

        for h in range(N_DEV - 1):
            r_ = (h + 1) % 2
            for l in range(LANES):
                for d in (0, 1):
                    ag_fly[(d, l)].wait()
                    if h < N_DEV - 2:
                        rd = make_rdma(ag_buf, ag_sem, d, l, h + 1)
                        rd.start()
                        ag_fly[(d, l)] = rd
                    org = ag_origin(d, h)
                    out_store((h + 1) % 2, d, l, org,
                              ag_buf[d][r_, l].astype(jnp.float32))

        for cp in out_fly.values():
            cp.wait()

    return pl.pallas_call(
        body,
        out_shape=jax.ShapeDtypeStruct((M, D), jnp.float32),
        in_specs=[
            pl.BlockSpec(memory_space=pl.ANY),
            pl.BlockSpec(memory_space=pl.ANY),
            pl.BlockSpec(memory_space=pltpu.VMEM),
        ],
        out_specs=pl.BlockSpec(memory_space=pl.ANY),
        scratch_shapes=[
            pltpu.VMEM((2, LANES, S, D), jnp.bfloat16),
            pltpu.VMEM((2, LANES, S, D), jnp.bfloat16),
            pltpu.VMEM((2, LANES, S, D), jnp.bfloat16),
            pltpu.VMEM((2, LANES, S, D), jnp.bfloat16),
            pltpu.VMEM((C, D), jnp.float32),
            pltpu.VMEM((M, D), jnp.float32),
            pltpu.VMEM((2, 2, LANES, S, D), jnp.float32),
            pltpu.SemaphoreType.DMA((2, LANES)),
            pltpu.SemaphoreType.DMA((2, LANES)),
            pltpu.SemaphoreType.DMA((2, LANES)),
            pltpu.SemaphoreType.DMA((2, LANES)),
            pltpu.SemaphoreType.DMA((2, LANES)),
            pltpu.SemaphoreType.DMA((2, LANES)),
            pltpu.SemaphoreType.DMA((2, LANES)),
            pltpu.SemaphoreType.DMA((2, LANES)),
            pltpu.SemaphoreType.DMA,
            pltpu.SemaphoreType.DMA((4,)),
            pltpu.SemaphoreType.DMA((2, 2, LANES)),
        ],
        compiler_params=pltpu.CompilerParams(
            collective_id=0, vmem_limit_bytes=100 * 1024 * 1024),
    )(partial, resid, gamma)
